# Initial kernel scaffold; baseline (speedup 1.0000x reference)
import jax
import jax.numpy as jnp
from jax.experimental import pallas as pl


def kernel(x, edge_index, W1, b1, W2, b2):
    raise NotImplementedError("write your pallas kernel here")


# speedup vs baseline: 6.6330x; 6.6330x over previous
import functools

import jax
import jax.numpy as jnp
from jax import lax
from jax.experimental import pallas as pl
from jax.experimental.pallas import tpu as pltpu
from jax.experimental.pallas import tpu_sc as plsc

NC = 2
NS = 16
NW = NC * NS
CH = 128
LANES = 16

_MESH = plsc.VectorSubcoreMesh(
    core_axis_name="c", subcore_axis_name="s", num_cores=NC, num_subcores=NS
)


def _sc_deg(dstp, ones, zrow, n_acc, epw):
    nch = epw // CH
    stripe = n_acc // NS

    def body(dstp_hbm, ones_hbm, z_hbm, out_hbm, didx, ones_v, stage_v, deg_sh):
        cid = lax.axis_index("c")
        sid = lax.axis_index("s")
        wid = sid * NC + cid
        base = wid * epw
        pltpu.sync_copy(z_hbm, stage_v)
        pltpu.sync_copy(stage_v, deg_sh.at[pl.ds(sid * stripe, stripe)])
        pltpu.sync_copy(ones_hbm, ones_v)
        plsc.subcore_barrier()

        def step(i, carry):
            off = pl.multiple_of(base + i * CH, CH)
            pltpu.sync_copy(dstp_hbm.at[pl.ds(off, CH)], didx.at[0])
            pltpu.sync_copy(ones_v, deg_sh.at[didx.at[0]], add=True)
            return carry

        lax.fori_loop(0, nch, step, 0)
        plsc.subcore_barrier()
        pltpu.sync_copy(deg_sh.at[pl.ds(sid * stripe, stripe)], stage_v)
        pltpu.sync_copy(
            stage_v,
            out_hbm.at[pl.ds(cid * n_acc + sid * stripe, stripe)],
        )

    return pl.kernel(
        body,
        out_type=jax.ShapeDtypeStruct((NC * n_acc,), jnp.float32),
        mesh=_MESH,
        scratch_types=[
            pltpu.VMEM((1, CH), jnp.int32),
            pltpu.VMEM((CH,), jnp.float32),
            pltpu.VMEM((stripe,), jnp.float32),
            pltpu.VMEM_SHARED((n_acc,), jnp.float32),
        ],
    )(dstp, ones, zrow)


def _sc_scatter(g, srcp, dstp, zblk, n_acc, epw):
    n, d = g.shape
    nch = epw // CH
    stripe = n_acc // NS

    def body(g_hbm, srcp_hbm, dstp_hbm, z_hbm, out_hbm, sidx, didx, rows, acc_sh, sem):
        cid = lax.axis_index("c")
        sid = lax.axis_index("s")
        wid = sid * NC + cid
        base = wid * epw
        pltpu.sync_copy(z_hbm, acc_sh.at[pl.ds(sid * stripe, stripe)])
        plsc.subcore_barrier()

        def step(i, carry):
            off = pl.multiple_of(base + i * CH, CH)
            pltpu.sync_copy(srcp_hbm.at[pl.ds(off, CH)], sidx.at[0])
            pltpu.sync_copy(dstp_hbm.at[pl.ds(off, CH)], didx.at[0])
            pltpu.async_copy(g_hbm.at[sidx.at[0]], rows, sem).wait()
            pltpu.sync_copy(rows, acc_sh.at[didx.at[0]], add=True)
            return carry

        lax.fori_loop(0, nch, step, 0)
        plsc.subcore_barrier()
        pltpu.sync_copy(
            acc_sh.at[pl.ds(sid * stripe, stripe)],
            out_hbm.at[cid, pl.ds(sid * stripe, stripe)],
        )

    return pl.kernel(
        body,
        out_type=jax.ShapeDtypeStruct((NC, n_acc, d), jnp.float32),
        mesh=_MESH,
        scratch_types=[
            pltpu.VMEM((1, CH), jnp.int32),
            pltpu.VMEM((1, CH), jnp.int32),
            pltpu.VMEM((CH, d), jnp.float32),
            pltpu.VMEM_SHARED((n_acc, d), jnp.float32),
            pltpu.SemaphoreType.DMA,
        ],
    )(g, srcp, dstp, zblk)


def _sc_edge_dot(h2, srcp, dstp, epw, e_pad):
    n, d = h2.shape
    nch = epw // CH
    nv = d // LANES

    def body(h_hbm, s_hbm, t_hbm, out_hbm, sidx, didx, rs, rd, dots, sem):
        cid = lax.axis_index("c")
        sid = lax.axis_index("s")
        wid = sid * NC + cid
        base = wid * epw

        def step(i, carry):
            off = pl.multiple_of(base + i * CH, CH)
            pltpu.sync_copy(s_hbm.at[pl.ds(off, CH)], sidx.at[0])
            pltpu.sync_copy(t_hbm.at[pl.ds(off, CH)], didx.at[0])
            pltpu.async_copy(h_hbm.at[sidx.at[0]], rs, sem).wait()
            pltpu.async_copy(h_hbm.at[didx.at[0]], rd, sem).wait()

            lane = lax.iota(jnp.int32, LANES)
            for gi in range(CH // LANES):
                def edot(k, vec):
                    e = gi * LANES + k
                    v = rs[e, pl.ds(0, LANES)] * rd[e, pl.ds(0, LANES)]
                    for j in range(1, nv):
                        v = v + rs[e, pl.ds(j * LANES, LANES)] * rd[e, pl.ds(j * LANES, LANES)]
                    for sh in (8, 4, 2, 1):
                        v = v + jnp.take_along_axis(
                            v, lane ^ sh, axis=0, mode="promise_in_bounds")
                    return jnp.where(lane == k, v, vec)

                dots[pl.ds(gi * LANES, LANES)] = lax.fori_loop(
                    0, LANES, edot, jnp.zeros((LANES,), jnp.float32))
            pltpu.sync_copy(dots, out_hbm.at[pl.ds(off, CH)])
            return carry

        lax.fori_loop(0, nch, step, 0)

    return pl.kernel(
        body,
        out_type=jax.ShapeDtypeStruct((e_pad,), jnp.float32),
        mesh=_MESH,
        scratch_types=[
            pltpu.VMEM((1, CH), jnp.int32),
            pltpu.VMEM((1, CH), jnp.int32),
            pltpu.VMEM((CH, d), jnp.float32),
            pltpu.VMEM((CH, d), jnp.float32),
            pltpu.VMEM((CH,), jnp.float32),
            pltpu.SemaphoreType.DMA,
        ],
    )(h2, srcp, dstp)


def _tc_layer1(deg3, x, w1, n):
    d = x.shape[1]

    def body(deg_ref, x_ref, w_ref, dinv_ref, g_ref):
        dv = lax.rsqrt(deg_ref[0] + deg_ref[1] + 1.0)[:n]
        h = jnp.dot(x_ref[...], w_ref[...], preferred_element_type=jnp.float32,
                    precision=lax.Precision.HIGHEST)
        dinv_ref[...] = dv
        g_ref[...] = h * dv

    return pl.pallas_call(
        body,
        out_shape=(
            jax.ShapeDtypeStruct((n, 1), jnp.float32),
            jax.ShapeDtypeStruct((n, d), jnp.float32),
        ),
    )(deg3, x, w1)


def _tc_layer2(s1, g1, dinv, b1, w2, n):
    d = g1.shape[1]

    def body(s_ref, g_ref, dinv_ref, b_ref, w_ref, g2_ref):
        s = s_ref[0, :n, :] + s_ref[1, :n, :] + g_ref[...]
        z = jnp.maximum(dinv_ref[...] * s + b_ref[...], 0.0)
        h = jnp.dot(z, w_ref[...], preferred_element_type=jnp.float32,
                    precision=lax.Precision.HIGHEST)
        g2_ref[...] = h * dinv_ref[...]

    return pl.pallas_call(
        body,
        out_shape=jax.ShapeDtypeStruct((n, d), jnp.float32),
    )(s1, g1, dinv, b1, w2)


def _tc_layer3(s2, g2, dinv, b2, n):
    d = g2.shape[1]

    def body(s_ref, g_ref, dinv_ref, b_ref, h_ref):
        s = s_ref[0, :n, :] + s_ref[1, :n, :] + g_ref[...]
        h_ref[...] = dinv_ref[...] * s + b_ref[...]

    return pl.pallas_call(
        body,
        out_shape=jax.ShapeDtypeStruct((n, d), jnp.float32),
    )(s2, g2, dinv, b2)


def kernel(x, edge_index, W1, b1, W2, b2):
    n, d = x.shape
    e = edge_index.shape[1]
    e_pad = -(-e // (NW * CH)) * (NW * CH)
    epw = e_pad // NW
    n_acc = -(-(n + 1) // CH) * CH
    stripe = n_acc // NS
    pad = e_pad - e

    src = edge_index[0]
    dst = edge_index[1]
    srcp = jnp.concatenate([src, jnp.zeros((pad,), jnp.int32)])
    dstp_sc = jnp.concatenate([dst, jnp.full((pad,), n, jnp.int32)])
    dstp_g = jnp.concatenate([dst, jnp.zeros((pad,), jnp.int32)])
    ones = jnp.ones((CH,), jnp.float32)
    zrow = jnp.zeros((stripe,), jnp.float32)
    zblk = jnp.zeros((stripe, d), jnp.float32)

    deg = _sc_deg(dstp_sc, ones, zrow, n_acc, epw)
    dinv, g1 = _tc_layer1(deg.reshape(NC, n_acc, 1), x, W1, n)
    s1 = _sc_scatter(g1, srcp, dstp_sc, zblk, n_acc, epw)
    g2 = _tc_layer2(s1, g1, dinv, b1.reshape(1, d), W2, n)
    s2 = _sc_scatter(g2, srcp, dstp_sc, zblk, n_acc, epw)
    h2 = _tc_layer3(s2, g2, dinv, b2.reshape(1, d), n)
    logits = _sc_edge_dot(h2, srcp, dstp_g, epw, e_pad)
    return logits[:e]


# speedup vs baseline: 6.7861x; 1.0231x over previous
import functools

import jax
import jax.numpy as jnp
from jax import lax
from jax.experimental import pallas as pl
from jax.experimental.pallas import tpu as pltpu
from jax.experimental.pallas import tpu_sc as plsc

NC = 2
NS = 16
NW = NC * NS
CH = 128
NB = 4
NB2 = 2
LANES = 16

_MESH = plsc.VectorSubcoreMesh(
    core_axis_name="c", subcore_axis_name="s", num_cores=NC, num_subcores=NS
)


def _sc_deg(dstp3, ones, zrow, n_acc, epw):
    nch = epw // CH
    stripe = n_acc // NS

    def body(dstp_hbm, ones_hbm, z_hbm, out_hbm, didx, ones_v, stage_v, deg_sh, sem):
        cid = lax.axis_index("c")
        sid = lax.axis_index("s")
        wid = sid * NC + cid
        pltpu.sync_copy(z_hbm, stage_v)
        pltpu.sync_copy(stage_v, deg_sh.at[pl.ds(sid * stripe, stripe)])
        pltpu.sync_copy(dstp_hbm.at[wid], didx)
        pltpu.sync_copy(ones_hbm, ones_v)
        plsc.subcore_barrier()

        def fire(i, c):
            pltpu.async_copy(ones_v, deg_sh.at[didx.at[i]], sem, add=True)
            return c

        lax.fori_loop(0, nch, fire, 0)

        def drain(i, c):
            pltpu.make_async_copy(ones_v, deg_sh.at[didx.at[i]], sem).wait()
            return c

        lax.fori_loop(0, nch, drain, 0)
        plsc.subcore_barrier()
        pltpu.sync_copy(deg_sh.at[pl.ds(sid * stripe, stripe)], stage_v)
        pltpu.sync_copy(
            stage_v,
            out_hbm.at[pl.ds(cid * n_acc + sid * stripe, stripe)],
        )

    return pl.kernel(
        body,
        out_type=jax.ShapeDtypeStruct((NC * n_acc,), jnp.float32),
        mesh=_MESH,
        scratch_types=[
            pltpu.VMEM((nch, CH), jnp.int32),
            pltpu.VMEM((CH,), jnp.float32),
            pltpu.VMEM((stripe,), jnp.float32),
            pltpu.VMEM_SHARED((n_acc,), jnp.float32),
            pltpu.SemaphoreType.DMA,
        ],
    )(dstp3, ones, zrow)


@functools.lru_cache(maxsize=None)
def _sc_scatter_kernel(n, d, n_acc, epw):
    nch = epw // CH
    SB = 8
    nsc = nch // SB
    npair = nsc // 2
    stripe = n_acc // NS

    def body(g_hbm, srcp_hbm, dstp_hbm, z_hbm, out_hbm, sidx, didx, rows, acc_sh, *sems):
        gsems, ssems, isems_s, isems_d = sems[:2], sems[2:4], sems[4:6], sems[6:8]
        cid = lax.axis_index("c")
        sid = lax.axis_index("s")
        wid = sid * NC + cid
        pltpu.sync_copy(z_hbm, acc_sh.at[pl.ds(sid * stripe, stripe)])
        for sl in range(2):
            pltpu.async_copy(srcp_hbm.at[wid, pl.ds(sl * SB, SB)], sidx.at[sl], isems_s[sl])
            pltpu.async_copy(dstp_hbm.at[wid, pl.ds(sl * SB, SB)], didx.at[sl], isems_d[sl])
        pltpu.make_async_copy(srcp_hbm.at[wid, pl.ds(0, SB)], sidx.at[0], isems_s[0]).wait()
        pltpu.make_async_copy(dstp_hbm.at[wid, pl.ds(0, SB)], didx.at[0], isems_d[0]).wait()
        pltpu.async_copy(g_hbm.at[sidx.at[0, 0]], rows.at[pl.ds(0, CH)], gsems[0])
        plsc.subcore_barrier()

        def chunk(s0, ph, b, first):
            rs = b % 2
            cur = rows.at[pl.ds(rs * CH, CH)]
            nxt = rows.at[pl.ds((1 - rs) * CH, CH)]
            pltpu.make_async_copy(g_hbm.at[sidx.at[ph, b]], cur, gsems[rs]).wait()
            pltpu.async_copy(cur, acc_sh.at[didx.at[ph, b]], ssems[rs], add=True)
            if not first:
                pidx = didx.at[ph, b - 1] if b > 0 else didx.at[1 - ph, SB - 1]
                pltpu.make_async_copy(nxt, acc_sh.at[pidx], ssems[1 - rs]).wait()
                if b == 0:
                    sr = lax.rem(s0 + 1, nsc)
                    pltpu.async_copy(
                        srcp_hbm.at[wid, pl.ds(sr * SB, SB)], sidx.at[1 - ph], isems_s[1 - ph])
                    pltpu.async_copy(
                        dstp_hbm.at[wid, pl.ds(sr * SB, SB)], didx.at[1 - ph], isems_d[1 - ph])
            if b == SB - 1:
                srn = lax.rem(s0 + 1, nsc)
                pltpu.make_async_copy(
                    srcp_hbm.at[wid, pl.ds(srn * SB, SB)], sidx.at[1 - ph], isems_s[1 - ph]).wait()
                pltpu.make_async_copy(
                    dstp_hbm.at[wid, pl.ds(srn * SB, SB)], didx.at[1 - ph], isems_d[1 - ph]).wait()
                nref = sidx.at[1 - ph, 0]
            else:
                nref = sidx.at[ph, b + 1]
            pltpu.async_copy(g_hbm.at[nref], nxt, gsems[1 - rs])

        for ph in range(2):
            for b in range(SB):
                chunk(ph, ph, b, ph == 0 and b == 0)

        def pair(p, carry):
            for ph in range(2):
                for b in range(SB):
                    chunk(p * 2 + ph, ph, b, False)
            return carry

        lax.fori_loop(1, npair, pair, 0)
        pltpu.make_async_copy(
            rows.at[pl.ds(CH, CH)], acc_sh.at[didx.at[1, SB - 1]], ssems[1]).wait()
        pltpu.make_async_copy(
            g_hbm.at[sidx.at[0, 0]], rows.at[pl.ds(0, CH)], gsems[0]).wait()
        plsc.subcore_barrier()
        pltpu.sync_copy(
            acc_sh.at[pl.ds(sid * stripe, stripe)],
            out_hbm.at[cid, pl.ds(sid * stripe, stripe)],
        )

    return pl.kernel(
        body,
        out_type=jax.ShapeDtypeStruct((NC, n_acc, d), jnp.float32),
        mesh=_MESH,
        scratch_types=[
            pltpu.VMEM((2, SB, CH), jnp.int32),
            pltpu.VMEM((2, SB, CH), jnp.int32),
            pltpu.VMEM((2 * CH, d), jnp.float32),
            pltpu.VMEM_SHARED((n_acc, d), jnp.float32),
        ] + [pltpu.SemaphoreType.DMA] * 8,
    )


def _sc_scatter(g, srcp3, dstp3, zblk, n_acc, epw):
    n, d = g.shape
    return _sc_scatter_kernel(n, d, n_acc, epw)(g, srcp3, dstp3, zblk)


def _sc_edge_dot(h2, srcp3, dstp3, epw, e_pad):
    n, d = h2.shape
    nch = epw // CH
    nblk = nch // NB2
    nv = d // LANES

    def body(h_hbm, s_hbm, t_hbm, out_hbm, sidx, didx, rs, rd, dots, *sems):
        gs, gd, osems = sems[:NB2], sems[NB2:2 * NB2], sems[2 * NB2:]
        cid = lax.axis_index("c")
        sid = lax.axis_index("s")
        wid = sid * NC + cid
        obase = wid * epw
        pltpu.sync_copy(s_hbm.at[wid], sidx)
        pltpu.sync_copy(t_hbm.at[wid], didx)
        for b in range(NB2):
            pltpu.async_copy(h_hbm.at[sidx.at[b]], rs.at[pl.ds(b * CH, CH)], gs[b])
            pltpu.async_copy(h_hbm.at[didx.at[b]], rd.at[pl.ds(b * CH, CH)], gd[b])
        lane = lax.iota(jnp.int32, LANES)

        def chunk(i, b, first):
            bufs = rs.at[pl.ds(b * CH, CH)]
            bufd = rd.at[pl.ds(b * CH, CH)]
            pltpu.make_async_copy(h_hbm.at[sidx.at[i]], bufs, gs[b]).wait()
            pltpu.make_async_copy(h_hbm.at[didx.at[i]], bufd, gd[b]).wait()
            if not first:
                pltpu.make_async_copy(
                    dots.at[pl.ds(b * CH, CH)],
                    out_hbm.at[pl.ds(obase + (i - NB2) * CH, CH)],
                    osems[b],
                ).wait()
            for gi in range(CH // LANES):
                def edot(k, vec):
                    e = b * CH + gi * LANES + k
                    v = rs[e, pl.ds(0, LANES)] * rd[e, pl.ds(0, LANES)]
                    for jj in range(1, nv):
                        v = v + rs[e, pl.ds(jj * LANES, LANES)] * rd[e, pl.ds(jj * LANES, LANES)]
                    for sh in (8, 4, 2, 1):
                        v = v + jnp.take_along_axis(
                            v, lane ^ sh, axis=0, mode="promise_in_bounds")
                    return jnp.where(lane == k, v, vec)

                dots[pl.ds(b * CH + gi * LANES, LANES)] = lax.fori_loop(
                    0, LANES, edot, jnp.zeros((LANES,), jnp.float32))
            pltpu.async_copy(
                dots.at[pl.ds(b * CH, CH)],
                out_hbm.at[pl.ds(obase + i * CH, CH)],
                osems[b],
            )
            j = lax.rem(i + NB2, nch)
            pltpu.async_copy(h_hbm.at[sidx.at[j]], bufs, gs[b])
            pltpu.async_copy(h_hbm.at[didx.at[j]], bufd, gd[b])

        for b in range(NB2):
            chunk(b, b, True)

        def block(i0, carry):
            for b in range(NB2):
                chunk(i0 * NB2 + b, b, False)
            return carry

        lax.fori_loop(1, nblk, block, 0)
        for b in range(NB2):
            i = (nblk - 1) * NB2 + b
            pltpu.make_async_copy(
                dots.at[pl.ds(b * CH, CH)],
                out_hbm.at[pl.ds(obase + i * CH, CH)],
                osems[b],
            ).wait()
            pltpu.make_async_copy(h_hbm.at[sidx.at[b]], rs.at[pl.ds(b * CH, CH)], gs[b]).wait()
            pltpu.make_async_copy(h_hbm.at[didx.at[b]], rd.at[pl.ds(b * CH, CH)], gd[b]).wait()

    return pl.kernel(
        body,
        out_type=jax.ShapeDtypeStruct((e_pad,), jnp.float32),
        mesh=_MESH,
        scratch_types=[
            pltpu.VMEM((nch, CH), jnp.int32),
            pltpu.VMEM((nch, CH), jnp.int32),
            pltpu.VMEM((NB2 * CH, d), jnp.float32),
            pltpu.VMEM((NB2 * CH, d), jnp.float32),
            pltpu.VMEM((NB2 * CH,), jnp.float32),
        ] + [pltpu.SemaphoreType.DMA] * (3 * NB2),
    )(h2, srcp3, dstp3)


def _tc_layer1(deg3, x, w1, n):
    d = x.shape[1]

    def body(deg_ref, x_ref, w_ref, dinv_ref, g_ref):
        dv = lax.rsqrt(deg_ref[0] + deg_ref[1] + 1.0)[:n]
        h = jnp.dot(x_ref[...], w_ref[...], preferred_element_type=jnp.float32,
                    precision=lax.Precision.HIGHEST)
        dinv_ref[...] = dv
        g_ref[...] = h * dv

    return pl.pallas_call(
        body,
        out_shape=(
            jax.ShapeDtypeStruct((n, 1), jnp.float32),
            jax.ShapeDtypeStruct((n, d), jnp.float32),
        ),
    )(deg3, x, w1)


def _tc_layer2(s1, g1, dinv, b1, w2, n):
    d = g1.shape[1]

    def body(s_ref, g_ref, dinv_ref, b_ref, w_ref, g2_ref):
        s = s_ref[0, :n, :] + s_ref[1, :n, :] + g_ref[...]
        z = jnp.maximum(dinv_ref[...] * s + b_ref[...], 0.0)
        h = jnp.dot(z, w_ref[...], preferred_element_type=jnp.float32,
                    precision=lax.Precision.HIGHEST)
        g2_ref[...] = h * dinv_ref[...]

    return pl.pallas_call(
        body,
        out_shape=jax.ShapeDtypeStruct((n, d), jnp.float32),
    )(s1, g1, dinv, b1, w2)


def _tc_layer3(s2, g2, dinv, b2, n):
    d = g2.shape[1]

    def body(s_ref, g_ref, dinv_ref, b_ref, h_ref):
        s = s_ref[0, :n, :] + s_ref[1, :n, :] + g_ref[...]
        h_ref[...] = dinv_ref[...] * s + b_ref[...]

    return pl.pallas_call(
        body,
        out_shape=jax.ShapeDtypeStruct((n, d), jnp.float32),
    )(s2, g2, dinv, b2)


def kernel(x, edge_index, W1, b1, W2, b2):
    n, d = x.shape
    e = edge_index.shape[1]
    grain = NW * CH * 16
    e_pad = -(-e // grain) * grain
    epw = e_pad // NW
    nch = epw // CH
    n_acc = -(-(n + 1) // CH) * CH
    stripe = n_acc // NS
    pad = e_pad - e

    src = edge_index[0]
    dst = edge_index[1]
    srcp3 = jnp.concatenate([src, jnp.zeros((pad,), jnp.int32)]).reshape(NW, nch, CH)
    dstp3_sc = jnp.concatenate([dst, jnp.full((pad,), n, jnp.int32)]).reshape(NW, nch, CH)
    dstp3_g = jnp.concatenate([dst, jnp.zeros((pad,), jnp.int32)]).reshape(NW, nch, CH)
    ones = jnp.ones((CH,), jnp.float32)
    zrow = jnp.zeros((stripe,), jnp.float32)
    zblk = jnp.zeros((stripe, d), jnp.float32)

    deg = _sc_deg(dstp3_sc, ones, zrow, n_acc, epw)
    dinv, g1 = _tc_layer1(deg.reshape(NC, n_acc, 1), x, W1, n)
    s1 = _sc_scatter(g1, srcp3, dstp3_sc, zblk, n_acc, epw)
    g2 = _tc_layer2(s1, g1, dinv, b1.reshape(1, d), W2, n)
    s2 = _sc_scatter(g2, srcp3, dstp3_sc, zblk, n_acc, epw)
    h2 = _tc_layer3(s2, g2, dinv, b2.reshape(1, d), n)
    logits = _sc_edge_dot(h2, srcp3, dstp3_g, epw, e_pad)
    return logits[:e]
